# initial kernel scaffold (unmeasured)
import jax
import jax.numpy as jnp
from jax import lax
from jax.experimental import pallas as pl
from jax.experimental.pallas import tpu as pltpu

N_DEV = 16
B, SQ, D = 1, 256, 1024
SKV, HQ, DH = 4096, 8, 128
CHUNK = 2048
SCALE = 0.08838834764831843
NEG_INF = -1e30


def kernel(x, Wq, Wo, K_ext, V_ext):
    def body(x_ref, wq_ref, wo_ref, k_ref, v_ref, out_ref,
             blk, ml, fin, ssem, rsem, ssem_ml, rsem_ml, credit, exit_sem):
        my = lax.axis_index("i")
        left = lax.rem(my + N_DEV - 1, N_DEV)
        right = lax.rem(my + 1, N_DEV)

        barrier_sem = pltpu.get_barrier_semaphore()
        for nbr in (left, right):
            pl.semaphore_signal(
                barrier_sem, inc=1,
                device_id=(nbr,), device_id_type=pl.DeviceIdType.MESH,
            )
        pl.semaphore_wait(barrier_sem, 2)

        xs = x_ref[0]
        for h in range(HQ):
            wq_h = wq_ref[:, h * DH:(h + 1) * DH]
            blk[0, h, 0] = lax.dot_general(
                wq_h, xs, (((0,), (1,)), ((), ())),
                preferred_element_type=jnp.float32,
            )
            blk[0, h, 1] = jnp.zeros((DH, SQ), jnp.float32)
            ml[0, h, 0] = jnp.full((1, SQ), NEG_INF, jnp.float32)
            ml[0, h, 1] = jnp.zeros((1, SQ), jnp.float32)

        def compute(slot):
            for h in range(HQ):
                q = blk[slot, h, 0]
                acc = blk[slot, h, 1]
                m = ml[slot, h, 0]
                l = ml[slot, h, 1]
                for c in range(SKV // CHUNK):
                    kc = k_ref[0, c * CHUNK:(c + 1) * CHUNK, h, :]
                    s = lax.dot_general(
                        kc, q, (((1,), (0,)), ((), ())),
                        preferred_element_type=jnp.float32,
                    ) * SCALE
                    m_new = jnp.maximum(m, jnp.max(s, axis=0, keepdims=True))
                    alpha = jnp.exp(m - m_new)
                    p = jnp.exp(s - m_new)
                    vc = v_ref[0, c * CHUNK:(c + 1) * CHUNK, h, :]
                    pv = lax.dot_general(
                        vc, p, (((0,), (0,)), ((), ())),
                        preferred_element_type=jnp.float32,
                    )
                    acc = acc * alpha + pv
                    l = l * alpha + jnp.sum(p, axis=0, keepdims=True)
                    m = m_new
                blk[slot, h, 1] = acc
                ml[slot, h, 0] = m
                ml[slot, h, 1] = l

        def hop(idx, slot, nslot):
            r1 = pltpu.make_async_remote_copy(
                src_ref=blk.at[slot], dst_ref=blk.at[nslot],
                send_sem=ssem.at[idx], recv_sem=rsem.at[idx],
                device_id=(right,), device_id_type=pl.DeviceIdType.MESH,
            )
            r2 = pltpu.make_async_remote_copy(
                src_ref=ml.at[slot], dst_ref=ml.at[nslot],
                send_sem=ssem_ml.at[idx], recv_sem=rsem_ml.at[idx],
                device_id=(right,), device_id_type=pl.DeviceIdType.MESH,
            )
            r1.start()
            r2.start()
            r1.wait()
            r2.wait()

        def grant_credit():
            pl.semaphore_signal(
                credit, inc=1,
                device_id=(left,), device_id_type=pl.DeviceIdType.MESH,
            )

        compute(0)
        hop(0, 0, 1)
        grant_credit()

        def step(h, _):
            slot = lax.rem(h, 2)
            nslot = 1 - slot
            compute(slot)
            pl.semaphore_wait(credit, 1)
            hop(h, slot, nslot)

            @pl.when(h <= N_DEV - 3)
            def _():
                grant_credit()

            return _

        lax.fori_loop(1, N_DEV - 1, step, None)

        compute(1)
        out_final = jnp.zeros((SQ, D), jnp.float32)
        for h in range(HQ):
            a_h = blk[1, h, 1] / ml[1, h, 1]
            out_final = out_final + lax.dot_general(
                a_h, wo_ref[h * DH:(h + 1) * DH, :], (((0,), (0,)), ((), ())),
                preferred_element_type=jnp.float32,
            )
        fin[:] = out_final
        rfin = pltpu.make_async_remote_copy(
            src_ref=fin, dst_ref=out_ref.at[0],
            send_sem=ssem.at[N_DEV - 1], recv_sem=rsem.at[N_DEV - 1],
            device_id=(right,), device_id_type=pl.DeviceIdType.MESH,
        )
        rfin.start()
        rfin.wait()

        for nbr in (left, right):
            pl.semaphore_signal(
                exit_sem, inc=1,
                device_id=(nbr,), device_id_type=pl.DeviceIdType.MESH,
            )
        pl.semaphore_wait(exit_sem, 2)

    return pl.pallas_call(
        body,
        out_shape=jax.ShapeDtypeStruct((B, SQ, D), jnp.float32),
        in_specs=[pl.BlockSpec(memory_space=pltpu.VMEM)] * 5,
        out_specs=pl.BlockSpec(memory_space=pltpu.VMEM),
        scratch_shapes=[
            pltpu.VMEM((2, HQ, 2, DH, SQ), jnp.float32),
            pltpu.VMEM((2, HQ, 2, 1, SQ), jnp.float32),
            pltpu.VMEM((SQ, D), jnp.float32),
            pltpu.SemaphoreType.DMA((N_DEV,)),
            pltpu.SemaphoreType.DMA((N_DEV,)),
            pltpu.SemaphoreType.DMA((N_DEV,)),
            pltpu.SemaphoreType.DMA((N_DEV,)),
            pltpu.SemaphoreType.REGULAR,
            pltpu.SemaphoreType.REGULAR,
        ],
        compiler_params=pltpu.CompilerParams(collective_id=0),
    )(x, Wq, Wo, K_ext, V_ext)


# baseline (device time: 875611 ns/iter reference)
import jax
import jax.numpy as jnp
from jax import lax
from jax.experimental import pallas as pl
from jax.experimental.pallas import tpu as pltpu

N_DEV = 16
B, SQ, D = 1, 256, 1024
SKV, HQ, DH = 4096, 8, 128
CHUNK = 1024
SCALE = 0.08838834764831843
NEG_INF = -1e30


def kernel(x, Wq, Wo, K_ext, V_ext):
    q0 = jnp.transpose((x[0] @ Wq).reshape(SQ, HQ, DH), (1, 2, 0))
    k = jnp.swapaxes(K_ext[0], 0, 1)
    v = jnp.swapaxes(V_ext[0], 0, 1)

    def body(q_ref, k_ref, v_ref, out_ref,
             blk, ml, fin, ssem, rsem, ssem_ml, rsem_ml, credit, exit_sem):
        my = lax.axis_index("i")
        left = lax.rem(my + N_DEV - 1, N_DEV)
        right = lax.rem(my + 1, N_DEV)

        barrier_sem = pltpu.get_barrier_semaphore()
        for nbr in (left, right):
            pl.semaphore_signal(
                barrier_sem, inc=1,
                device_id=(nbr,), device_id_type=pl.DeviceIdType.MESH,
            )
        pl.semaphore_wait(barrier_sem, 2)

        def init_head(h, _):
            blk[0, h, 0] = q_ref[h]
            blk[0, h, 1] = jnp.zeros((DH, SQ), jnp.float32)
            ml[0, h, 0] = jnp.full((1, SQ), NEG_INF, jnp.float32)
            ml[0, h, 1] = jnp.zeros((1, SQ), jnp.float32)
            return _

        lax.fori_loop(0, HQ, init_head, None)

        def compute(slot):
            def head(h, _):
                q = blk[slot, h, 0]

                def chunk_step(c, carry):
                    acc, m, l = carry
                    kc = k_ref[h, pl.ds(c * CHUNK, CHUNK), :]
                    s = lax.dot_general(
                        kc, q, (((1,), (0,)), ((), ())),
                        preferred_element_type=jnp.float32,
                    ) * SCALE
                    m_new = jnp.maximum(m, jnp.max(s, axis=0, keepdims=True))
                    alpha = jnp.exp(m - m_new)
                    p = jnp.exp(s - m_new)
                    vc = v_ref[h, pl.ds(c * CHUNK, CHUNK), :]
                    pv = lax.dot_general(
                        vc, p, (((0,), (0,)), ((), ())),
                        preferred_element_type=jnp.float32,
                    )
                    return (acc * alpha + pv,
                            m_new,
                            l * alpha + jnp.sum(p, axis=0, keepdims=True))

                acc, m, l = lax.fori_loop(
                    0, SKV // CHUNK, chunk_step,
                    (blk[slot, h, 1], ml[slot, h, 0], ml[slot, h, 1]),
                )
                blk[slot, h, 1] = acc
                ml[slot, h, 0] = m
                ml[slot, h, 1] = l
                return _

            lax.fori_loop(0, HQ, head, None)

        def hop(idx, slot, nslot):
            r1 = pltpu.make_async_remote_copy(
                src_ref=blk.at[slot], dst_ref=blk.at[nslot],
                send_sem=ssem.at[idx], recv_sem=rsem.at[idx],
                device_id=(right,), device_id_type=pl.DeviceIdType.MESH,
            )
            r2 = pltpu.make_async_remote_copy(
                src_ref=ml.at[slot], dst_ref=ml.at[nslot],
                send_sem=ssem_ml.at[idx], recv_sem=rsem_ml.at[idx],
                device_id=(right,), device_id_type=pl.DeviceIdType.MESH,
            )
            r1.start()
            r2.start()
            r1.wait()
            r2.wait()

        def grant_credit():
            pl.semaphore_signal(
                credit, inc=1,
                device_id=(left,), device_id_type=pl.DeviceIdType.MESH,
            )

        compute(0)
        hop(0, 0, 1)
        grant_credit()

        def step(h, _):
            slot = lax.rem(h, 2)
            nslot = 1 - slot
            compute(slot)
            pl.semaphore_wait(credit, 1)
            hop(h, slot, nslot)

            @pl.when(h <= N_DEV - 3)
            def _():
                grant_credit()

            return _

        lax.fori_loop(1, N_DEV - 1, step, None)

        compute(1)

        def normalize(h, _):
            fin[h] = blk[1, h, 1] / ml[1, h, 1]
            return _

        lax.fori_loop(0, HQ, normalize, None)

        rfin = pltpu.make_async_remote_copy(
            src_ref=fin, dst_ref=out_ref,
            send_sem=ssem.at[N_DEV - 1], recv_sem=rsem.at[N_DEV - 1],
            device_id=(right,), device_id_type=pl.DeviceIdType.MESH,
        )
        rfin.start()
        rfin.wait()

        for nbr in (left, right):
            pl.semaphore_signal(
                exit_sem, inc=1,
                device_id=(nbr,), device_id_type=pl.DeviceIdType.MESH,
            )
        pl.semaphore_wait(exit_sem, 2)

    attn = pl.pallas_call(
        body,
        out_shape=jax.ShapeDtypeStruct((HQ, DH, SQ), jnp.float32),
        in_specs=[pl.BlockSpec(memory_space=pltpu.VMEM)] * 3,
        out_specs=pl.BlockSpec(memory_space=pltpu.VMEM),
        scratch_shapes=[
            pltpu.VMEM((2, HQ, 2, DH, SQ), jnp.float32),
            pltpu.VMEM((2, HQ, 2, 1, SQ), jnp.float32),
            pltpu.VMEM((HQ, DH, SQ), jnp.float32),
            pltpu.SemaphoreType.DMA((N_DEV,)),
            pltpu.SemaphoreType.DMA((N_DEV,)),
            pltpu.SemaphoreType.DMA((N_DEV,)),
            pltpu.SemaphoreType.DMA((N_DEV,)),
            pltpu.SemaphoreType.REGULAR,
            pltpu.SemaphoreType.REGULAR,
        ],
        compiler_params=pltpu.CompilerParams(collective_id=0),
    )(q0, k, v)

    out = jnp.transpose(attn, (2, 0, 1)).reshape(SQ, HQ * DH) @ Wo
    return out[None]


# device time: 632967 ns/iter; 1.3833x vs baseline; 1.3833x over previous
import jax
import jax.numpy as jnp
from jax import lax
from jax.experimental import pallas as pl
from jax.experimental.pallas import tpu as pltpu

N_DEV = 16
B, SQ, D = 1, 256, 1024
SKV, HQ, DH = 4096, 8, 128
CHUNK = 1024
ROWS = DH + 8
SCALE = 0.08838834764831843
NEG_INF = -1e30


def kernel(x, Wq, Wo, K_ext, V_ext):
    q0 = jnp.transpose((x[0] @ Wq).reshape(SQ, HQ, DH), (1, 2, 0))
    k = jnp.swapaxes(K_ext[0], 0, 1)
    v = jnp.swapaxes(V_ext[0], 0, 1)

    def body(q_ref, k_ref, v_ref, out_ref,
             blk, fin, ssem, rsem, fssem, frsem, credit, exit_sem):
        my = lax.axis_index("i")
        left = lax.rem(my + N_DEV - 1, N_DEV)
        right = lax.rem(my + 1, N_DEV)

        barrier_sem = pltpu.get_barrier_semaphore()
        for nbr in (left, right):
            pl.semaphore_signal(
                barrier_sem, inc=1,
                device_id=(nbr,), device_id_type=pl.DeviceIdType.MESH,
            )
        pl.semaphore_wait(barrier_sem, 2)

        def compute_head(slot, h):
            qfull = blk[slot, h, 0]
            q = qfull[0:DH]

            def chunk_step(c, carry):
                acc, m, l = carry
                kc = k_ref[h, pl.ds(c * CHUNK, CHUNK), :]
                s = lax.dot_general(
                    kc, q, (((1,), (0,)), ((), ())),
                    preferred_element_type=jnp.float32,
                ) * SCALE
                m_new = jnp.maximum(m, jnp.max(s, axis=0, keepdims=True))
                alpha = jnp.exp(m - m_new)
                p = jnp.exp(s - m_new)
                vc = v_ref[h, pl.ds(c * CHUNK, CHUNK), :]
                pv = lax.dot_general(
                    vc, p, (((0,), (0,)), ((), ())),
                    preferred_element_type=jnp.float32,
                )
                return (acc * alpha + pv,
                        m_new,
                        l * alpha + jnp.sum(p, axis=0, keepdims=True))

            acc, m, l = lax.fori_loop(
                0, SKV // CHUNK, chunk_step,
                (blk[slot, h, 1, 0:DH], qfull[DH:DH + 1], qfull[DH + 1:DH + 2]),
            )
            blk[slot, h, 1, 0:DH] = acc
            blk[slot, h, 0, DH:DH + 2] = jnp.concatenate([m, l], axis=0)

        def send_desc(s, d, h):
            return pltpu.make_async_remote_copy(
                src_ref=blk.at[s, h], dst_ref=blk.at[d, h],
                send_sem=ssem.at[d, h], recv_sem=rsem.at[d, h],
                device_id=(right,), device_id_type=pl.DeviceIdType.MESH,
            )

        def recv_desc(s, h):
            return pltpu.make_async_remote_copy(
                src_ref=blk.at[1 - s, h], dst_ref=blk.at[s, h],
                send_sem=ssem.at[s, h], recv_sem=rsem.at[s, h],
                device_id=(left,), device_id_type=pl.DeviceIdType.MESH,
            )

        def drain_desc(s, h):
            return pltpu.make_async_remote_copy(
                src_ref=blk.at[1 - s, h], dst_ref=blk.at[s, h],
                send_sem=ssem.at[s, h], recv_sem=rsem.at[s, h],
                device_id=(right,), device_id_type=pl.DeviceIdType.MESH,
            )

        def step0_head(h, _):
            blk[0, h, 0, 0:DH] = q_ref[h]
            blk[0, h, 0, DH:DH + 1] = jnp.full((1, SQ), NEG_INF, jnp.float32)
            blk[0, h, 0, DH + 1:DH + 2] = jnp.zeros((1, SQ), jnp.float32)
            blk[0, h, 1] = jnp.zeros((ROWS, SQ), jnp.float32)
            compute_head(0, h)
            send_desc(0, 1, h).start()
            return _

        lax.fori_loop(0, HQ, step0_head, None)

        def step(i, _):
            s = lax.rem(i, 2)
            d = 1 - s

            def head(h, __):
                recv_desc(s, h).wait_recv()
                compute_head(s, h)
                drain_desc(s, h).wait_send()
                pl.semaphore_signal(
                    credit, inc=1,
                    device_id=(left,), device_id_type=pl.DeviceIdType.MESH,
                )
                pl.semaphore_wait(credit, 1)
                send_desc(s, d, h).start()
                return __

            lax.fori_loop(0, HQ, head, None)
            return _

        lax.fori_loop(1, N_DEV - 1, step, None)

        def step15_head(h, _):
            recv_desc(1, h).wait_recv()
            compute_head(1, h)
            drain_desc(1, h).wait_send()
            fin[h] = blk[1, h, 1, 0:DH] / blk[1, h, 0, DH + 1:DH + 2]
            return _

        lax.fori_loop(0, HQ, step15_head, None)

        rfin = pltpu.make_async_remote_copy(
            src_ref=fin, dst_ref=out_ref,
            send_sem=fssem, recv_sem=frsem,
            device_id=(right,), device_id_type=pl.DeviceIdType.MESH,
        )
        rfin.start()
        rfin.wait()

        for nbr in (left, right):
            pl.semaphore_signal(
                exit_sem, inc=1,
                device_id=(nbr,), device_id_type=pl.DeviceIdType.MESH,
            )
        pl.semaphore_wait(exit_sem, 2)

    attn = pl.pallas_call(
        body,
        out_shape=jax.ShapeDtypeStruct((HQ, DH, SQ), jnp.float32),
        in_specs=[pl.BlockSpec(memory_space=pltpu.VMEM)] * 3,
        out_specs=pl.BlockSpec(memory_space=pltpu.VMEM),
        scratch_shapes=[
            pltpu.VMEM((2, HQ, 2, ROWS, SQ), jnp.float32),
            pltpu.VMEM((HQ, DH, SQ), jnp.float32),
            pltpu.SemaphoreType.DMA((2, HQ)),
            pltpu.SemaphoreType.DMA((2, HQ)),
            pltpu.SemaphoreType.DMA,
            pltpu.SemaphoreType.DMA,
            pltpu.SemaphoreType.REGULAR,
            pltpu.SemaphoreType.REGULAR,
        ],
        compiler_params=pltpu.CompilerParams(collective_id=0),
    )(q0, k, v)

    out = jnp.transpose(attn, (2, 0, 1)).reshape(SQ, HQ * DH) @ Wo
    return out[None]


# device time: 622177 ns/iter; 1.4073x vs baseline; 1.0173x over previous
import jax
import jax.numpy as jnp
from jax import lax
from jax.experimental import pallas as pl
from jax.experimental.pallas import tpu as pltpu

N_DEV = 16
B, SQ, D = 1, 256, 1024
SKV, HQ, DH = 4096, 8, 128
CHUNK = 1024
ROWS = DH + 8
SCALE = 0.08838834764831843
NEG_INF = -1e30


def kernel(x, Wq, Wo, K_ext, V_ext):
    q0 = jnp.transpose(
        (x[0] @ (Wq * SCALE)).reshape(SQ, HQ, DH), (1, 2, 0))
    k = jnp.swapaxes(K_ext[0], 0, 1)
    v = jnp.swapaxes(V_ext[0], 0, 1)

    def body(q_ref, k_ref, v_ref, out_ref,
             blk, fin, ssem, rsem, fssem, frsem, credit, exit_sem):
        my = lax.axis_index("i")
        left = lax.rem(my + N_DEV - 1, N_DEV)
        right = lax.rem(my + 1, N_DEV)

        barrier_sem = pltpu.get_barrier_semaphore()
        for nbr in (left, right):
            pl.semaphore_signal(
                barrier_sem, inc=1,
                device_id=(nbr,), device_id_type=pl.DeviceIdType.MESH,
            )
        pl.semaphore_wait(barrier_sem, 2)

        def compute_head(slot, h):
            qfull = blk[slot, h, 0]
            q = qfull[0:DH]

            def chunk_step(c, carry):
                acc, m, l = carry
                kc = k_ref[h, pl.ds(c * CHUNK, CHUNK), :]
                s = lax.dot_general(
                    kc, q, (((1,), (0,)), ((), ())),
                    preferred_element_type=jnp.float32,
                )
                m_new = jnp.maximum(m, jnp.max(s, axis=0, keepdims=True))
                alpha = jnp.exp(m - m_new)
                p = jnp.exp(s - m_new)
                vc = v_ref[h, pl.ds(c * CHUNK, CHUNK), :]
                pv = lax.dot_general(
                    vc, p, (((0,), (0,)), ((), ())),
                    preferred_element_type=jnp.float32,
                )
                return (acc * alpha + pv,
                        m_new,
                        l * alpha + jnp.sum(p, axis=0, keepdims=True))

            acc, m, l = lax.fori_loop(
                0, SKV // CHUNK, chunk_step,
                (blk[slot, h, 1, 0:DH], qfull[DH:DH + 1], qfull[DH + 1:DH + 2]),
            )
            blk[slot, h, 1, 0:DH] = acc
            blk[slot, h, 0, DH:DH + 2] = jnp.concatenate([m, l], axis=0)

        def send_desc(s, d, h):
            return pltpu.make_async_remote_copy(
                src_ref=blk.at[s, h], dst_ref=blk.at[d, h],
                send_sem=ssem.at[d, h], recv_sem=rsem.at[d, h],
                device_id=(right,), device_id_type=pl.DeviceIdType.MESH,
            )

        def recv_desc(s, h):
            return pltpu.make_async_remote_copy(
                src_ref=blk.at[1 - s, h], dst_ref=blk.at[s, h],
                send_sem=ssem.at[s, h], recv_sem=rsem.at[s, h],
                device_id=(left,), device_id_type=pl.DeviceIdType.MESH,
            )

        def drain_desc(s, h):
            return pltpu.make_async_remote_copy(
                src_ref=blk.at[1 - s, h], dst_ref=blk.at[s, h],
                send_sem=ssem.at[s, h], recv_sem=rsem.at[s, h],
                device_id=(right,), device_id_type=pl.DeviceIdType.MESH,
            )

        def step0_head(h, _):
            blk[0, h, 0, 0:DH] = q_ref[h]
            blk[0, h, 0, DH:DH + 1] = jnp.full((1, SQ), NEG_INF, jnp.float32)
            blk[0, h, 0, DH + 1:DH + 2] = jnp.zeros((1, SQ), jnp.float32)
            blk[0, h, 1] = jnp.zeros((ROWS, SQ), jnp.float32)
            compute_head(0, h)
            send_desc(0, 1, h).start()
            return _

        lax.fori_loop(0, HQ, step0_head, None)

        def step(i, _):
            s = lax.rem(i, 2)
            d = 1 - s

            def head(h, __):
                recv_desc(s, h).wait_recv()
                compute_head(s, h)
                drain_desc(s, h).wait_send()
                pl.semaphore_signal(
                    credit, inc=1,
                    device_id=(left,), device_id_type=pl.DeviceIdType.MESH,
                )
                pl.semaphore_wait(credit, 1)
                send_desc(s, d, h).start()
                return __

            lax.fori_loop(0, HQ, head, None)
            return _

        lax.fori_loop(1, N_DEV - 1, step, None)

        def step15_head(h, _):
            recv_desc(1, h).wait_recv()
            compute_head(1, h)
            drain_desc(1, h).wait_send()
            fin[h] = blk[1, h, 1, 0:DH] / blk[1, h, 0, DH + 1:DH + 2]
            return _

        lax.fori_loop(0, HQ, step15_head, None)

        rfin = pltpu.make_async_remote_copy(
            src_ref=fin, dst_ref=out_ref,
            send_sem=fssem, recv_sem=frsem,
            device_id=(right,), device_id_type=pl.DeviceIdType.MESH,
        )
        rfin.start()
        rfin.wait()

        for nbr in (left, right):
            pl.semaphore_signal(
                exit_sem, inc=1,
                device_id=(nbr,), device_id_type=pl.DeviceIdType.MESH,
            )
        pl.semaphore_wait(exit_sem, 2)

    attn = pl.pallas_call(
        body,
        out_shape=jax.ShapeDtypeStruct((HQ, DH, SQ), jnp.float32),
        in_specs=[pl.BlockSpec(memory_space=pltpu.VMEM)] * 3,
        out_specs=pl.BlockSpec(memory_space=pltpu.VMEM),
        scratch_shapes=[
            pltpu.VMEM((2, HQ, 2, ROWS, SQ), jnp.float32),
            pltpu.VMEM((HQ, DH, SQ), jnp.float32),
            pltpu.SemaphoreType.DMA((2, HQ)),
            pltpu.SemaphoreType.DMA((2, HQ)),
            pltpu.SemaphoreType.DMA,
            pltpu.SemaphoreType.DMA,
            pltpu.SemaphoreType.REGULAR,
            pltpu.SemaphoreType.REGULAR,
        ],
        compiler_params=pltpu.CompilerParams(collective_id=0),
    )(q0, k, v)

    out = jnp.transpose(attn, (2, 0, 1)).reshape(SQ, HQ * DH) @ Wo
    return out[None]


# device time: 516403 ns/iter; 1.6956x vs baseline; 1.2048x over previous
import jax
import jax.numpy as jnp
from jax import lax
from jax.experimental import pallas as pl
from jax.experimental.pallas import tpu as pltpu

N_DEV = 16
B, SQ, D = 1, 256, 1024
SKV, HQ, DH = 4096, 8, 128
CHUNK = 1024
ROWS = DH + 8
SCALE = 0.08838834764831843
NEG_INF = -1e30


def kernel(x, Wq, Wo, K_ext, V_ext):
    q0 = jnp.transpose(
        (x[0] @ (Wq * SCALE)).reshape(SQ, HQ, DH), (1, 2, 0))
    k = jnp.swapaxes(K_ext[0], 0, 1)
    v = jnp.swapaxes(V_ext[0], 0, 1)

    def body(q_ref, k_ref, v_ref, out_ref,
             blk, fin, ssem, rsem, fssem, frsem, credit, exit_sem):
        my = lax.axis_index("i")
        left = lax.rem(my + N_DEV - 1, N_DEV)
        right = lax.rem(my + 1, N_DEV)

        barrier_sem = pltpu.get_barrier_semaphore()
        for nbr in (left, right):
            pl.semaphore_signal(
                barrier_sem, inc=1,
                device_id=(nbr,), device_id_type=pl.DeviceIdType.MESH,
            )
        pl.semaphore_wait(barrier_sem, 2)

        def compute_pair(slot, h0, h1):
            qf0 = blk[slot, h0, 0]
            qf1 = blk[slot, h1, 0]
            q_0 = qf0[0:DH]
            q_1 = qf1[0:DH]

            def one(h, q, acc, m, l, c):
                kc = k_ref[h, pl.ds(c * CHUNK, CHUNK), :]
                s = lax.dot_general(
                    kc, q, (((1,), (0,)), ((), ())),
                    preferred_element_type=jnp.float32,
                )
                m_new = jnp.maximum(m, jnp.max(s, axis=0, keepdims=True))
                alpha = jnp.exp(m - m_new)
                p = jnp.exp(s - m_new)
                vc = v_ref[h, pl.ds(c * CHUNK, CHUNK), :]
                pv = lax.dot_general(
                    vc, p, (((0,), (0,)), ((), ())),
                    preferred_element_type=jnp.float32,
                )
                return (acc * alpha + pv,
                        m_new,
                        l * alpha + jnp.sum(p, axis=0, keepdims=True))

            def chunk_step(c, carry):
                a0, m0, l0, a1, m1, l1 = carry
                a0, m0, l0 = one(h0, q_0, a0, m0, l0, c)
                a1, m1, l1 = one(h1, q_1, a1, m1, l1, c)
                return (a0, m0, l0, a1, m1, l1)

            a0, m0, l0, a1, m1, l1 = lax.fori_loop(
                0, SKV // CHUNK, chunk_step,
                (blk[slot, h0, 1, 0:DH], qf0[DH:DH + 1], qf0[DH + 1:DH + 2],
                 blk[slot, h1, 1, 0:DH], qf1[DH:DH + 1], qf1[DH + 1:DH + 2]),
            )
            blk[slot, h0, 1, 0:DH] = a0
            blk[slot, h0, 0, DH:DH + 2] = jnp.concatenate([m0, l0], axis=0)
            blk[slot, h1, 1, 0:DH] = a1
            blk[slot, h1, 0, DH:DH + 2] = jnp.concatenate([m1, l1], axis=0)

        def send_desc(s, d, h):
            return pltpu.make_async_remote_copy(
                src_ref=blk.at[s, h], dst_ref=blk.at[d, h],
                send_sem=ssem.at[d, h], recv_sem=rsem.at[d, h],
                device_id=(right,), device_id_type=pl.DeviceIdType.MESH,
            )

        def recv_desc(s, h):
            return pltpu.make_async_remote_copy(
                src_ref=blk.at[1 - s, h], dst_ref=blk.at[s, h],
                send_sem=ssem.at[s, h], recv_sem=rsem.at[s, h],
                device_id=(left,), device_id_type=pl.DeviceIdType.MESH,
            )

        def drain_desc(s, h):
            return pltpu.make_async_remote_copy(
                src_ref=blk.at[1 - s, h], dst_ref=blk.at[s, h],
                send_sem=ssem.at[s, h], recv_sem=rsem.at[s, h],
                device_id=(right,), device_id_type=pl.DeviceIdType.MESH,
            )

        def step0_pair(hp, _):
            h0 = 2 * hp
            h1 = h0 + 1
            for h in (h0, h1):
                blk[0, h, 0, 0:DH] = q_ref[h]
                blk[0, h, 0, DH:DH + 1] = jnp.full((1, SQ), NEG_INF,
                                                   jnp.float32)
                blk[0, h, 0, DH + 1:DH + 2] = jnp.zeros((1, SQ), jnp.float32)
                blk[0, h, 1] = jnp.zeros((ROWS, SQ), jnp.float32)
            compute_pair(0, h0, h1)
            send_desc(0, 1, h0).start()
            send_desc(0, 1, h1).start()
            return _

        lax.fori_loop(0, HQ // 2, step0_pair, None)

        def step(i, _):
            s = lax.rem(i, 2)
            d = 1 - s

            def pair(hp, __):
                h0 = 2 * hp
                h1 = h0 + 1
                recv_desc(s, h0).wait_recv()
                recv_desc(s, h1).wait_recv()
                compute_pair(s, h0, h1)
                drain_desc(s, h0).wait_send()
                drain_desc(s, h1).wait_send()
                pl.semaphore_signal(
                    credit, inc=2,
                    device_id=(left,), device_id_type=pl.DeviceIdType.MESH,
                )
                pl.semaphore_wait(credit, 2)
                send_desc(s, d, h0).start()
                send_desc(s, d, h1).start()
                return __

            lax.fori_loop(0, HQ // 2, pair, None)
            return _

        lax.fori_loop(1, N_DEV - 1, step, None)

        def step15_pair(hp, _):
            h0 = 2 * hp
            h1 = h0 + 1
            recv_desc(1, h0).wait_recv()
            recv_desc(1, h1).wait_recv()
            compute_pair(1, h0, h1)
            drain_desc(1, h0).wait_send()
            drain_desc(1, h1).wait_send()
            fin[h0] = blk[1, h0, 1, 0:DH] / blk[1, h0, 0, DH + 1:DH + 2]
            fin[h1] = blk[1, h1, 1, 0:DH] / blk[1, h1, 0, DH + 1:DH + 2]
            return _

        lax.fori_loop(0, HQ // 2, step15_pair, None)

        rfin = pltpu.make_async_remote_copy(
            src_ref=fin, dst_ref=out_ref,
            send_sem=fssem, recv_sem=frsem,
            device_id=(right,), device_id_type=pl.DeviceIdType.MESH,
        )
        rfin.start()
        rfin.wait()

        for nbr in (left, right):
            pl.semaphore_signal(
                exit_sem, inc=1,
                device_id=(nbr,), device_id_type=pl.DeviceIdType.MESH,
            )
        pl.semaphore_wait(exit_sem, 2)

    attn = pl.pallas_call(
        body,
        out_shape=jax.ShapeDtypeStruct((HQ, DH, SQ), jnp.float32),
        in_specs=[pl.BlockSpec(memory_space=pltpu.VMEM)] * 3,
        out_specs=pl.BlockSpec(memory_space=pltpu.VMEM),
        scratch_shapes=[
            pltpu.VMEM((2, HQ, 2, ROWS, SQ), jnp.float32),
            pltpu.VMEM((HQ, DH, SQ), jnp.float32),
            pltpu.SemaphoreType.DMA((2, HQ)),
            pltpu.SemaphoreType.DMA((2, HQ)),
            pltpu.SemaphoreType.DMA,
            pltpu.SemaphoreType.DMA,
            pltpu.SemaphoreType.REGULAR,
            pltpu.SemaphoreType.REGULAR,
        ],
        compiler_params=pltpu.CompilerParams(collective_id=0),
    )(q0, k, v)

    out = jnp.transpose(attn, (2, 0, 1)).reshape(SQ, HQ * DH) @ Wo
    return out[None]
